# baseline (speedup 1.0000x reference)
import jax
import jax.numpy as jnp
from jax import lax
from jax.experimental import pallas as pl
from jax.experimental.pallas import tpu as pltpu
from jax.experimental.pallas import tpu_sc as plsc

B = 16384
F = 128
NC = 2
NS = 16
L = 16
NW = NC * NS
BPW = B // NW
CH = 64
NCHUNK = BPW // CH
NBUF = 2

SHIFTS = (8, 4, 2, 1)
BITREV = (0, 8, 4, 12, 2, 10, 6, 14, 1, 9, 5, 13, 3, 11, 7, 15)

_DN = lax.GatherDimensionNumbers(
    offset_dims=(), collapsed_slice_dims=(0,), start_index_map=(0,))


def _body(user_hbm, item_hbm, uf_hbm, if_hbm, out_hbm,
          uidx_v, iidx_v, ubufs, vbufs, outv, stage, usems, vsems):
    c = lax.axis_index("c")
    s = lax.axis_index("s")
    wid = s * NC + c
    base = wid * BPW

    pltpu.sync_copy(user_hbm.at[pl.ds(base, BPW)], uidx_v)
    pltpu.sync_copy(item_hbm.at[pl.ds(base, BPW)], iidx_v)

    lane = lax.iota(jnp.int32, L)
    sidx = {sh: (lane ^ sh)[:, None] for sh in SHIFTS}
    keep = {sh: (lane & sh) == 0 for sh in SHIFTS}

    def shuffle(x, sh):
        return lax.gather(x, sidx[sh], _DN, (1,),
                          mode=lax.GatherScatterMode.PROMISE_IN_BOUNDS)

    def merge(x, y, sh):
        return jnp.where(keep[sh], x + shuffle(x, sh), y + shuffle(y, sh))

    def start(co, b):
        pltpu.async_copy(
            uf_hbm.at[uidx_v.at[pl.ds(co * CH, CH)]], ubufs[b], usems[b])
        pltpu.async_copy(
            if_hbm.at[iidx_v.at[pl.ds(co * CH, CH)]], vbufs[b], vsems[b])

    for b in range(NBUF):
        start(b, b)

    def c_body(cc, _):
        for b in range(NBUF):
            co = cc * NBUF + b
            pltpu.make_async_copy(
                uf_hbm.at[uidx_v.at[pl.ds(0, CH)]], ubufs[b], usems[b]).wait()
            pltpu.make_async_copy(
                if_hbm.at[iidx_v.at[pl.ds(0, CH)]], vbufs[b], vsems[b]).wait()
            ubuf, vbuf = ubufs[b], vbufs[b]
            for g in range(CH // L):
                for j in range(L):
                    r = g * L + BITREV[j]
                    v = ubuf[r, pl.ds(0, L)] * vbuf[r, pl.ds(0, L)]
                    for k in range(1, F // L):
                        v = v + ubuf[r, pl.ds(k * L, L)] * vbuf[r, pl.ds(k * L, L)]
                    stage[pl.ds(j * L, L)] = v
                stack = []
                for j in range(L):
                    v = stage[pl.ds(j * L, L)]
                    lvl = 0
                    while stack and stack[-1][0] == lvl:
                        pv = stack.pop()[1]
                        v = merge(pv, v, SHIFTS[lvl])
                        lvl += 1
                    stack.append((lvl, v))
                outv[pl.ds(co * CH + g * L, L)] = stack[0][1]

            @pl.when(co + NBUF < NCHUNK)
            def _():
                start(co + NBUF, b)

        return 0

    lax.fori_loop(0, NCHUNK // NBUF, c_body, 0)

    pltpu.sync_copy(outv, out_hbm.at[pl.ds(base, BPW)])


def kernel(user, item, user_factors, item_factors):
    mesh = plsc.VectorSubcoreMesh(core_axis_name="c", subcore_axis_name="s")
    k = pl.kernel(
        _body,
        out_type=jax.ShapeDtypeStruct((B,), jnp.float32),
        mesh=mesh,
        scratch_types=[
            pltpu.VMEM((BPW,), jnp.int32),
            pltpu.VMEM((BPW,), jnp.int32),
            [pltpu.VMEM((CH, F), jnp.float32) for _ in range(NBUF)],
            [pltpu.VMEM((CH, F), jnp.float32) for _ in range(NBUF)],
            pltpu.VMEM((BPW,), jnp.float32),
            pltpu.VMEM((L * L,), jnp.float32),
            [pltpu.SemaphoreType.DMA for _ in range(NBUF)],
            [pltpu.SemaphoreType.DMA for _ in range(NBUF)],
        ],
    )
    return k(user, item, user_factors, item_factors)


# speedup vs baseline: 1.0136x; 1.0136x over previous
import jax
import jax.numpy as jnp
from jax import lax
from jax.experimental import pallas as pl
from jax.experimental.pallas import tpu as pltpu
from jax.experimental.pallas import tpu_sc as plsc

B = 16384
F = 128
NC = 2
NS = 16
L = 16
NW = NC * NS
BPW = B // NW
CH = 32
NCHUNK = BPW // CH
NBUF = 4

SHIFTS = (8, 4, 2, 1)
BITREV = (0, 8, 4, 12, 2, 10, 6, 14, 1, 9, 5, 13, 3, 11, 7, 15)

_DN = lax.GatherDimensionNumbers(
    offset_dims=(), collapsed_slice_dims=(0,), start_index_map=(0,))


def _body(user_hbm, item_hbm, uf_hbm, if_hbm, out_hbm,
          uidx_v, iidx_v, ubufs, vbufs, outv, stage, usems, vsems):
    c = lax.axis_index("c")
    s = lax.axis_index("s")
    wid = s * NC + c
    base = wid * BPW

    pltpu.sync_copy(user_hbm.at[pl.ds(base, BPW)], uidx_v)
    pltpu.sync_copy(item_hbm.at[pl.ds(base, BPW)], iidx_v)

    lane = lax.iota(jnp.int32, L)
    sidx = {sh: (lane ^ sh)[:, None] for sh in SHIFTS}
    keep = {sh: (lane & sh) == 0 for sh in SHIFTS}

    def shuffle(x, sh):
        return lax.gather(x, sidx[sh], _DN, (1,),
                          mode=lax.GatherScatterMode.PROMISE_IN_BOUNDS)

    def merge(x, y, sh):
        return jnp.where(keep[sh], x + shuffle(x, sh), y + shuffle(y, sh))

    def start(co, b):
        pltpu.async_copy(
            uf_hbm.at[uidx_v.at[pl.ds(co * CH, CH)]], ubufs[b], usems[b])
        pltpu.async_copy(
            if_hbm.at[iidx_v.at[pl.ds(co * CH, CH)]], vbufs[b], vsems[b])

    for b in range(NBUF):
        start(b, b)

    def c_body(cc, _):
        for b in range(NBUF):
            co = cc * NBUF + b
            pltpu.make_async_copy(
                uf_hbm.at[uidx_v.at[pl.ds(0, CH)]], ubufs[b], usems[b]).wait()
            pltpu.make_async_copy(
                if_hbm.at[iidx_v.at[pl.ds(0, CH)]], vbufs[b], vsems[b]).wait()
            ubuf, vbuf = ubufs[b], vbufs[b]
            for g in range(CH // L):
                for j in range(L):
                    r = g * L + BITREV[j]
                    v = ubuf[r, pl.ds(0, L)] * vbuf[r, pl.ds(0, L)]
                    for k in range(1, F // L):
                        v = v + ubuf[r, pl.ds(k * L, L)] * vbuf[r, pl.ds(k * L, L)]
                    stage[pl.ds(j * L, L)] = v
                stack = []
                for j in range(L):
                    v = stage[pl.ds(j * L, L)]
                    lvl = 0
                    while stack and stack[-1][0] == lvl:
                        pv = stack.pop()[1]
                        v = merge(pv, v, SHIFTS[lvl])
                        lvl += 1
                    stack.append((lvl, v))
                outv[pl.ds(co * CH + g * L, L)] = stack[0][1]

            @pl.when(co + NBUF < NCHUNK)
            def _():
                start(co + NBUF, b)

        return 0

    lax.fori_loop(0, NCHUNK // NBUF, c_body, 0)

    pltpu.sync_copy(outv, out_hbm.at[pl.ds(base, BPW)])


def kernel(user, item, user_factors, item_factors):
    mesh = plsc.VectorSubcoreMesh(core_axis_name="c", subcore_axis_name="s")
    k = pl.kernel(
        _body,
        out_type=jax.ShapeDtypeStruct((B,), jnp.float32),
        mesh=mesh,
        scratch_types=[
            pltpu.VMEM((BPW,), jnp.int32),
            pltpu.VMEM((BPW,), jnp.int32),
            [pltpu.VMEM((CH, F), jnp.float32) for _ in range(NBUF)],
            [pltpu.VMEM((CH, F), jnp.float32) for _ in range(NBUF)],
            pltpu.VMEM((BPW,), jnp.float32),
            pltpu.VMEM((L * L,), jnp.float32),
            [pltpu.SemaphoreType.DMA for _ in range(NBUF)],
            [pltpu.SemaphoreType.DMA for _ in range(NBUF)],
        ],
    )
    return k(user, item, user_factors, item_factors)


# speedup vs baseline: 1.2334x; 1.2168x over previous
import jax
import jax.numpy as jnp
from jax import lax
from jax.experimental import pallas as pl
from jax.experimental.pallas import tpu as pltpu
from jax.experimental.pallas import tpu_sc as plsc

B = 16384
F = 128
NC = 2
NS = 16
L = 16
NW = NC * NS
BPW = B // NW
CH = 16
NCHUNK = BPW // CH
NBUF = 2

SHIFTS = (8, 4, 2, 1)
BITREV = (0, 8, 4, 12, 2, 10, 6, 14, 1, 9, 5, 13, 3, 11, 7, 15)

_DN = lax.GatherDimensionNumbers(
    offset_dims=(), collapsed_slice_dims=(0,), start_index_map=(0,))


def _body(user_hbm, item_hbm, uf_hbm, if_hbm, out_hbm,
          uidx_v, iidx_v, ubufs, vbufs, outv, stage, usems, vsems):
    c = lax.axis_index("c")
    s = lax.axis_index("s")
    wid = s * NC + c
    base = wid * BPW

    pltpu.sync_copy(user_hbm.at[pl.ds(base, BPW)], uidx_v)
    pltpu.sync_copy(item_hbm.at[pl.ds(base, BPW)], iidx_v)

    lane = lax.iota(jnp.int32, L)
    sidx = {sh: (lane ^ sh)[:, None] for sh in SHIFTS}
    keep = {sh: (lane & sh) == 0 for sh in SHIFTS}

    def shuffle(x, sh):
        return lax.gather(x, sidx[sh], _DN, (1,),
                          mode=lax.GatherScatterMode.PROMISE_IN_BOUNDS)

    def merge(x, y, sh):
        return jnp.where(keep[sh], x + shuffle(x, sh), y + shuffle(y, sh))

    def start(co, b):
        pltpu.async_copy(
            uf_hbm.at[uidx_v.at[pl.ds(co * CH, CH)]], ubufs[b], usems[b])
        pltpu.async_copy(
            if_hbm.at[iidx_v.at[pl.ds(co * CH, CH)]], vbufs[b], vsems[b])

    for b in range(NBUF):
        start(b, b)

    def c_body(cc, _):
        for b in range(NBUF):
            co = cc * NBUF + b
            pltpu.make_async_copy(
                uf_hbm.at[uidx_v.at[pl.ds(0, CH)]], ubufs[b], usems[b]).wait()
            pltpu.make_async_copy(
                if_hbm.at[iidx_v.at[pl.ds(0, CH)]], vbufs[b], vsems[b]).wait()
            ubuf, vbuf = ubufs[b], vbufs[b]
            for g in range(CH // L):
                for j in range(L):
                    r = g * L + BITREV[j]
                    v = ubuf[r, pl.ds(0, L)] * vbuf[r, pl.ds(0, L)]
                    for k in range(1, F // L):
                        v = v + ubuf[r, pl.ds(k * L, L)] * vbuf[r, pl.ds(k * L, L)]
                    stage[pl.ds(j * L, L)] = v
                stack = []
                for j in range(L):
                    v = stage[pl.ds(j * L, L)]
                    lvl = 0
                    while stack and stack[-1][0] == lvl:
                        pv = stack.pop()[1]
                        v = merge(pv, v, SHIFTS[lvl])
                        lvl += 1
                    stack.append((lvl, v))
                outv[pl.ds(co * CH + g * L, L)] = stack[0][1]

            @pl.when(co + NBUF < NCHUNK)
            def _():
                start(co + NBUF, b)

        return 0

    lax.fori_loop(0, NCHUNK // NBUF, c_body, 0)

    pltpu.sync_copy(outv, out_hbm.at[pl.ds(base, BPW)])


def kernel(user, item, user_factors, item_factors):
    mesh = plsc.VectorSubcoreMesh(core_axis_name="c", subcore_axis_name="s")
    k = pl.kernel(
        _body,
        out_type=jax.ShapeDtypeStruct((B,), jnp.float32),
        mesh=mesh,
        scratch_types=[
            pltpu.VMEM((BPW,), jnp.int32),
            pltpu.VMEM((BPW,), jnp.int32),
            [pltpu.VMEM((CH, F), jnp.float32) for _ in range(NBUF)],
            [pltpu.VMEM((CH, F), jnp.float32) for _ in range(NBUF)],
            pltpu.VMEM((BPW,), jnp.float32),
            pltpu.VMEM((L * L,), jnp.float32),
            [pltpu.SemaphoreType.DMA for _ in range(NBUF)],
            [pltpu.SemaphoreType.DMA for _ in range(NBUF)],
        ],
    )
    return k(user, item, user_factors, item_factors)
